# baseline (speedup 1.0000x reference)
import functools

import jax
import jax.numpy as jnp
from jax import lax
from jax.experimental import pallas as pl
from jax.experimental.pallas import tpu as pltpu
from jax.experimental.pallas import tpu_sc as plsc

NBINS = 64
NC = 2
NS = 16
L = 16
NW = NC * NS


def _mm_body(c_ref, p_ref, a_ref, mm_ref, sc_ref):
    i = pl.program_id(0)
    a = lax.dot_general(
        c_ref[...], p_ref[...], (((1,), (1,)), ((), ())),
        preferred_element_type=jnp.float32)
    a_ref[...] = a
    bmin = jnp.min(a)
    bmax = jnp.max(a)

    @pl.when(i == 0)
    def _():
        sc_ref[0] = bmin
        sc_ref[1] = bmax

    @pl.when(i > 0)
    def _():
        sc_ref[0] = jnp.minimum(sc_ref[0], bmin)
        sc_ref[1] = jnp.maximum(sc_ref[1], bmax)

    mm_ref[0:1, :] = jnp.full((1, L), sc_ref[0], jnp.float32)
    mm_ref[1:2, :] = jnp.full((1, L), sc_ref[1], jnp.float32)


def _make_sc_hist(n):
    rows_per_w = n // NW
    ch_rows = min(8, rows_per_w)
    nch = rows_per_w // ch_rows
    mesh = plsc.VectorSubcoreMesh(core_axis_name="c", subcore_axis_name="s")

    @functools.partial(
        pl.kernel,
        mesh=mesh,
        compiler_params=pltpu.CompilerParams(needs_layout_passes=False),
        out_type=jax.ShapeDtypeStruct((NW, NBINS + 1, L), jnp.float32),
        scratch_types=[
            pltpu.VMEM((ch_rows, n), jnp.float32),
            pltpu.VMEM((ch_rows, n), jnp.float32),
            pltpu.VMEM((2, L), jnp.float32),
            pltpu.VMEM((NBINS + 1, L), jnp.float32),
            pltpu.VMEM((NBINS + 1, L), jnp.float32),
            pltpu.VMEM((NBINS + 1, L), jnp.float32),
            pltpu.VMEM((NBINS + 1, L), jnp.float32),
            pltpu.SemaphoreType.DMA,
            pltpu.SemaphoreType.DMA,
        ],
    )
    def sc_hist(a_hbm, mm_hbm, part_hbm, buf0, buf1, mmv, table, table2,
                table3, table4, sem0, sem1):
        wid = lax.axis_index("s") * NC + lax.axis_index("c")
        row0 = wid * rows_per_w

        pltpu.sync_copy(mm_hbm, mmv)
        vmin = mmv[0, :]
        vmax = mmv[1, :]
        interval = (vmax - vmin) * jnp.float32(1.0 / NBINS)
        inv = jnp.where(interval > 0, 1.0 / interval,
                        jnp.zeros_like(interval))
        e_top = vmin + jnp.float32(NBINS) * interval

        lane = lax.iota(jnp.int32, L)
        ones = jnp.ones((L,), jnp.float32)
        zeros16 = jnp.zeros((L,), jnp.float32)
        for r in range(NBINS + 1):
            table[r, :] = zeros16
            table2[r, :] = zeros16
            table3[r, :] = zeros16
            table4[r, :] = zeros16

        def start(c, buf, sem):
            pltpu.async_copy(
                a_hbm.at[pl.ds(row0 + c * ch_rows, ch_rows), :], buf, sem)

        def wait(buf, sem):
            pltpu.make_async_copy(
                a_hbm.at[pl.ds(0, ch_rows), :], buf, sem).wait()

        def process(buf):
            tabs = (table, table2, table3, table4)
            for r in range(ch_rows):
                @plsc.parallel_loop(0, n // L, step=4, unroll=2)
                def _(j):
                    for q in range(4):
                        v = buf[r, pl.ds(j * L + q * L, L)]
                        t = (v - vmin) * inv
                        k = t.astype(jnp.int32)
                        plsc.addupdate_scatter(tabs[q], [k, lane], ones,
                                               mask=v <= e_top)

        start(0, buf0, sem0)
        if nch > 1:
            start(1, buf1, sem1)

        def pair(cc, carry):
            c = cc * 2
            wait(buf0, sem0)
            process(buf0)

            @pl.when(c + 2 < nch)
            def _():
                start(c + 2, buf0, sem0)

            wait(buf1, sem1)
            process(buf1)

            @pl.when(c + 3 < nch)
            def _():
                start(c + 3, buf1, sem1)

            return carry

        lax.fori_loop(0, nch // 2, pair, 0)

        for r in range(NBINS + 1):
            table[r, :] = ((table[r, :] + table2[r, :])
                           + (table3[r, :] + table4[r, :]))
        pltpu.sync_copy(table, part_hbm.at[wid])

    return sc_hist


def _fin_body(part_ref, w_ref, mm_ref, out_ref, *, n):
    p3 = part_ref[...]
    s2 = jnp.sum(p3, axis=0)
    counts65 = jnp.sum(s2, axis=1)
    last = lax.iota(jnp.int32, NBINS) == (NBINS - 1)
    counts = counts65[:NBINS] + jnp.where(last, counts65[NBINS], 0.0)
    vmin = mm_ref[0, 0]
    vmax = mm_ref[1, 0]
    degenerate = (vmax - vmin) <= 0
    total = jnp.float32(n) * jnp.float32(n)
    counts = jnp.where(degenerate, jnp.where(last, total, 0.0), counts)
    hist = jax.nn.sigmoid(counts)
    beta = jnp.sum(hist * w_ref[0, :])
    out_ref[...] = jax.nn.sigmoid(beta).reshape(1, 1)


def kernel(current_feature, previous_feature, W):
    n, d = current_feature.shape
    br = min(1024, n)
    nb = n // br

    A, mm = pl.pallas_call(
        _mm_body,
        grid=(nb,),
        in_specs=[
            pl.BlockSpec((br, d), lambda i: (i, 0)),
            pl.BlockSpec((n, d), lambda i: (0, 0)),
        ],
        out_specs=[
            pl.BlockSpec((br, n), lambda i: (i, 0)),
            pl.BlockSpec((2, L), lambda i: (0, 0)),
        ],
        out_shape=[
            jax.ShapeDtypeStruct((n, n), jnp.float32),
            jax.ShapeDtypeStruct((2, L), jnp.float32),
        ],
        scratch_shapes=[pltpu.SMEM((2,), jnp.float32)],
    )(current_feature, previous_feature)

    partials = _make_sc_hist(n)(A, mm)

    out = pl.pallas_call(
        functools.partial(_fin_body, n=n),
        in_specs=[
            pl.BlockSpec((NW, NBINS + 1, L), lambda: (0, 0, 0)),
            pl.BlockSpec((1, NBINS), lambda: (0, 0)),
            pl.BlockSpec((2, L), lambda: (0, 0)),
        ],
        out_specs=pl.BlockSpec((1, 1), lambda: (0, 0)),
        out_shape=jax.ShapeDtypeStruct((1, 1), jnp.float32),
    )(partials, W, mm)
    return out.reshape(1)


# speedup vs baseline: 1.1675x; 1.1675x over previous
import functools

import jax
import jax.numpy as jnp
from jax import lax
from jax.experimental import pallas as pl
from jax.experimental.pallas import tpu as pltpu
from jax.experimental.pallas import tpu_sc as plsc

NBINS = 64
NC = 2
NS = 16
L = 16
NW = NC * NS


def _mm_body(c_ref, p_ref, a_ref, mm_ref, sc_ref):
    i = pl.program_id(0)
    a = lax.dot_general(
        c_ref[...], p_ref[...], (((1,), (1,)), ((), ())),
        preferred_element_type=jnp.float32)
    a_ref[...] = a
    bmin = jnp.min(a)
    bmax = jnp.max(a)

    @pl.when(i == 0)
    def _():
        sc_ref[0] = bmin
        sc_ref[1] = bmax

    @pl.when(i > 0)
    def _():
        sc_ref[0] = jnp.minimum(sc_ref[0], bmin)
        sc_ref[1] = jnp.maximum(sc_ref[1], bmax)

    mm_ref[0:1, :] = jnp.full((1, L), sc_ref[0], jnp.float32)
    mm_ref[1:2, :] = jnp.full((1, L), sc_ref[1], jnp.float32)


def _make_sc_hist(n):
    rows_per_w = n // NW
    ch_rows = min(8, rows_per_w)
    nch = rows_per_w // ch_rows
    mesh = plsc.VectorSubcoreMesh(core_axis_name="c", subcore_axis_name="s")

    @functools.partial(
        pl.kernel,
        mesh=mesh,
        compiler_params=pltpu.CompilerParams(needs_layout_passes=False),
        out_type=jax.ShapeDtypeStruct((NW, NBINS + 1, L), jnp.float32),
        scratch_types=[
            pltpu.VMEM((ch_rows, n), jnp.float32),
            pltpu.VMEM((ch_rows, n), jnp.float32),
            pltpu.VMEM((2, L), jnp.float32),
            pltpu.VMEM((NBINS + 1, L), jnp.float32),
            pltpu.VMEM((NBINS + 1, L), jnp.float32),
            pltpu.SemaphoreType.DMA,
            pltpu.SemaphoreType.DMA,
        ],
    )
    def sc_hist(a_hbm, mm_hbm, part_hbm, buf0, buf1, mmv, table, table2,
                sem0, sem1):
        wid = lax.axis_index("s") * NC + lax.axis_index("c")
        row0 = wid * rows_per_w

        pltpu.sync_copy(mm_hbm, mmv)
        vmin = mmv[0, :]
        vmax = mmv[1, :]
        interval = (vmax - vmin) * jnp.float32(1.0 / NBINS)
        inv = jnp.where(interval > 0, 1.0 / interval,
                        jnp.zeros_like(interval))
        e_top = vmin + jnp.float32(NBINS) * interval

        lane = lax.iota(jnp.int32, L)
        ones = jnp.ones((L,), jnp.float32)
        zeros16 = jnp.zeros((L,), jnp.float32)
        for r in range(NBINS + 1):
            table[r, :] = zeros16
            table2[r, :] = zeros16

        def start(c, buf, sem):
            pltpu.async_copy(
                a_hbm.at[pl.ds(row0 + c * ch_rows, ch_rows), :], buf, sem)

        def wait(buf, sem):
            pltpu.make_async_copy(
                a_hbm.at[pl.ds(0, ch_rows), :], buf, sem).wait()

        def process(buf):
            for r in range(ch_rows):
                @plsc.parallel_loop(0, n // L, step=2, unroll=8)
                def _(j):
                    v0 = buf[r, pl.ds(j * L, L)]
                    v1 = buf[r, pl.ds(j * L + L, L)]
                    t0 = (v0 - vmin) * inv
                    t1 = (v1 - vmin) * inv
                    k0 = t0.astype(jnp.int32)
                    k1 = t1.astype(jnp.int32)
                    plsc.addupdate_scatter(table, [k0, lane], ones,
                                           mask=v0 <= e_top)
                    plsc.addupdate_scatter(table2, [k1, lane], ones,
                                           mask=v1 <= e_top)

        start(0, buf0, sem0)
        if nch > 1:
            start(1, buf1, sem1)

        def pair(cc, carry):
            c = cc * 2
            wait(buf0, sem0)
            process(buf0)

            @pl.when(c + 2 < nch)
            def _():
                start(c + 2, buf0, sem0)

            wait(buf1, sem1)
            process(buf1)

            @pl.when(c + 3 < nch)
            def _():
                start(c + 3, buf1, sem1)

            return carry

        lax.fori_loop(0, nch // 2, pair, 0)

        for r in range(NBINS + 1):
            table[r, :] = table[r, :] + table2[r, :]
        pltpu.sync_copy(table, part_hbm.at[wid])

    return sc_hist


def _fin_body(part_ref, w_ref, mm_ref, out_ref, *, n):
    p3 = part_ref[...]
    s2 = jnp.sum(p3, axis=0)
    counts65 = jnp.sum(s2, axis=1)
    last = lax.iota(jnp.int32, NBINS) == (NBINS - 1)
    counts = counts65[:NBINS] + jnp.where(last, counts65[NBINS], 0.0)
    vmin = mm_ref[0, 0]
    vmax = mm_ref[1, 0]
    degenerate = (vmax - vmin) <= 0
    total = jnp.float32(n) * jnp.float32(n)
    counts = jnp.where(degenerate, jnp.where(last, total, 0.0), counts)
    hist = jax.nn.sigmoid(counts)
    beta = jnp.sum(hist * w_ref[0, :])
    out_ref[...] = jax.nn.sigmoid(beta).reshape(1, 1)


def kernel(current_feature, previous_feature, W):
    n, d = current_feature.shape
    br = min(512, n)
    nb = n // br

    A, mm = pl.pallas_call(
        _mm_body,
        grid=(nb,),
        in_specs=[
            pl.BlockSpec((br, d), lambda i: (i, 0)),
            pl.BlockSpec((n, d), lambda i: (0, 0)),
        ],
        out_specs=[
            pl.BlockSpec((br, n), lambda i: (i, 0)),
            pl.BlockSpec((2, L), lambda i: (0, 0)),
        ],
        out_shape=[
            jax.ShapeDtypeStruct((n, n), jnp.float32),
            jax.ShapeDtypeStruct((2, L), jnp.float32),
        ],
        scratch_shapes=[pltpu.SMEM((2,), jnp.float32)],
    )(current_feature, previous_feature)

    partials = _make_sc_hist(n)(A, mm)

    out = pl.pallas_call(
        functools.partial(_fin_body, n=n),
        in_specs=[
            pl.BlockSpec((NW, NBINS + 1, L), lambda: (0, 0, 0)),
            pl.BlockSpec((1, NBINS), lambda: (0, 0)),
            pl.BlockSpec((2, L), lambda: (0, 0)),
        ],
        out_specs=pl.BlockSpec((1, 1), lambda: (0, 0)),
        out_shape=jax.ShapeDtypeStruct((1, 1), jnp.float32),
    )(partials, W, mm)
    return out.reshape(1)


# speedup vs baseline: 1.2489x; 1.0697x over previous
import functools

import jax
import jax.numpy as jnp
from jax import lax
from jax.experimental import pallas as pl
from jax.experimental.pallas import tpu as pltpu
from jax.experimental.pallas import tpu_sc as plsc

NBINS = 64
NC = 2
NS = 16
L = 16
NW = NC * NS


def _mm_body(c_ref, p_ref, a_ref, mm_ref, sc_ref):
    i = pl.program_id(0)
    a = lax.dot_general(
        c_ref[...], p_ref[...], (((1,), (1,)), ((), ())),
        preferred_element_type=jnp.float32)
    a_ref[...] = a
    bmin = jnp.min(a)
    bmax = jnp.max(a)

    @pl.when(i == 0)
    def _():
        sc_ref[0] = bmin
        sc_ref[1] = bmax

    @pl.when(i > 0)
    def _():
        sc_ref[0] = jnp.minimum(sc_ref[0], bmin)
        sc_ref[1] = jnp.maximum(sc_ref[1], bmax)

    mm_ref[0:1, :] = jnp.full((1, L), sc_ref[0], jnp.float32)
    mm_ref[1:2, :] = jnp.full((1, L), sc_ref[1], jnp.float32)


def _make_sc_hist(n):
    rows_per_w = n // NW
    ch_rows = min(4, rows_per_w)
    nch = rows_per_w // ch_rows
    mesh = plsc.VectorSubcoreMesh(core_axis_name="c", subcore_axis_name="s")

    @functools.partial(
        pl.kernel,
        mesh=mesh,
        compiler_params=pltpu.CompilerParams(needs_layout_passes=False),
        out_type=jax.ShapeDtypeStruct((NW, NBINS + 1, L), jnp.float32),
        scratch_types=[
            pltpu.VMEM((ch_rows, n), jnp.float32),
            pltpu.VMEM((ch_rows, n), jnp.float32),
            pltpu.VMEM((2, L), jnp.float32),
            pltpu.VMEM((NBINS + 1, L), jnp.float32),
            pltpu.VMEM((NBINS + 1, L), jnp.float32),
            pltpu.SemaphoreType.DMA,
            pltpu.SemaphoreType.DMA,
        ],
    )
    def sc_hist(a_hbm, mm_hbm, part_hbm, buf0, buf1, mmv, table, table2,
                sem0, sem1):
        wid = lax.axis_index("s") * NC + lax.axis_index("c")
        row0 = wid * rows_per_w

        pltpu.sync_copy(mm_hbm, mmv)
        vmin = mmv[0, :]
        vmax = mmv[1, :]
        interval = (vmax - vmin) * jnp.float32(1.0 / NBINS)
        inv = jnp.where(interval > 0, 1.0 / interval,
                        jnp.zeros_like(interval))
        e_top = vmin + jnp.float32(NBINS) * interval

        lane = lax.iota(jnp.int32, L)
        ones = jnp.ones((L,), jnp.float32)
        zeros16 = jnp.zeros((L,), jnp.float32)
        for r in range(NBINS + 1):
            table[r, :] = zeros16
            table2[r, :] = zeros16

        def start(c, buf, sem):
            pltpu.async_copy(
                a_hbm.at[pl.ds(row0 + c * ch_rows, ch_rows), :], buf, sem)

        def wait(buf, sem):
            pltpu.make_async_copy(
                a_hbm.at[pl.ds(0, ch_rows), :], buf, sem).wait()

        def process(buf):
            for r in range(ch_rows):
                @plsc.parallel_loop(0, n // L, step=2, unroll=4)
                def _(j):
                    v0 = buf[r, pl.ds(j * L, L)]
                    v1 = buf[r, pl.ds(j * L + L, L)]
                    t0 = (v0 - vmin) * inv
                    t1 = (v1 - vmin) * inv
                    k0 = t0.astype(jnp.int32)
                    k1 = t1.astype(jnp.int32)
                    plsc.addupdate_scatter(table, [k0, lane], ones,
                                           mask=v0 <= e_top)
                    plsc.addupdate_scatter(table2, [k1, lane], ones,
                                           mask=v1 <= e_top)

        start(0, buf0, sem0)
        if nch > 1:
            start(1, buf1, sem1)

        def pair(cc, carry):
            c = cc * 2
            wait(buf0, sem0)
            process(buf0)

            @pl.when(c + 2 < nch)
            def _():
                start(c + 2, buf0, sem0)

            wait(buf1, sem1)
            process(buf1)

            @pl.when(c + 3 < nch)
            def _():
                start(c + 3, buf1, sem1)

            return carry

        lax.fori_loop(0, nch // 2, pair, 0)

        for r in range(NBINS + 1):
            table[r, :] = table[r, :] + table2[r, :]
        pltpu.sync_copy(table, part_hbm.at[wid])

    return sc_hist


def _fin_body(part_ref, w_ref, mm_ref, out_ref, *, n):
    p3 = part_ref[...]
    s2 = jnp.sum(p3, axis=0)
    counts65 = jnp.sum(s2, axis=1)
    last = lax.iota(jnp.int32, NBINS) == (NBINS - 1)
    counts = counts65[:NBINS] + jnp.where(last, counts65[NBINS], 0.0)
    vmin = mm_ref[0, 0]
    vmax = mm_ref[1, 0]
    degenerate = (vmax - vmin) <= 0
    total = jnp.float32(n) * jnp.float32(n)
    counts = jnp.where(degenerate, jnp.where(last, total, 0.0), counts)
    hist = jax.nn.sigmoid(counts)
    beta = jnp.sum(hist * w_ref[0, :])
    out_ref[...] = jax.nn.sigmoid(beta).reshape(1, 1)


def kernel(current_feature, previous_feature, W):
    n, d = current_feature.shape
    br = min(512, n)
    nb = n // br

    A, mm = pl.pallas_call(
        _mm_body,
        grid=(nb,),
        in_specs=[
            pl.BlockSpec((br, d), lambda i: (i, 0)),
            pl.BlockSpec((n, d), lambda i: (0, 0)),
        ],
        out_specs=[
            pl.BlockSpec((br, n), lambda i: (i, 0)),
            pl.BlockSpec((2, L), lambda i: (0, 0)),
        ],
        out_shape=[
            jax.ShapeDtypeStruct((n, n), jnp.float32),
            jax.ShapeDtypeStruct((2, L), jnp.float32),
        ],
        scratch_shapes=[pltpu.SMEM((2,), jnp.float32)],
    )(current_feature, previous_feature)

    partials = _make_sc_hist(n)(A, mm)

    out = pl.pallas_call(
        functools.partial(_fin_body, n=n),
        in_specs=[
            pl.BlockSpec((NW, NBINS + 1, L), lambda: (0, 0, 0)),
            pl.BlockSpec((1, NBINS), lambda: (0, 0)),
            pl.BlockSpec((2, L), lambda: (0, 0)),
        ],
        out_specs=pl.BlockSpec((1, 1), lambda: (0, 0)),
        out_shape=jax.ShapeDtypeStruct((1, 1), jnp.float32),
    )(partials, W, mm)
    return out.reshape(1)
